# baseline (speedup 1.0000x reference)
import functools

import jax
import jax.numpy as jnp
from jax import lax
from jax.experimental import pallas as pl
from jax.experimental.pallas import tpu as pltpu
from jax.experimental.pallas import tpu_sc as plsc

_NC = 2
_NS = 16
_NW = _NC * _NS
_L = 16

_CHUNK = 128
_NSLOT = 2


def _make_sc_gather(B, D, n_rows, n_cols, d_row):
    W = B // _NW
    n_chunks = W // _CHUNK
    d_col = D - d_row
    assert W % _CHUNK == 0 and n_chunks >= 2 * _NSLOT
    assert d_row % _L == 0 and d_col % _L == 0
    assert (n_chunks - _NSLOT) % _NSLOT == 0

    mesh = plsc.VectorSubcoreMesh(core_axis_name="c", subcore_axis_name="s")

    @functools.partial(
        pl.kernel,
        out_type=jax.ShapeDtypeStruct((B, D), jnp.float32),
        mesh=mesh,
        scratch_types=[
            pltpu.VMEM((n_rows * d_row,), jnp.float32),
            pltpu.VMEM((n_cols * d_col,), jnp.float32),
            pltpu.VMEM((W,), jnp.int32),
            pltpu.VMEM((W,), jnp.int32),
        ] + [pltpu.VMEM((_CHUNK, D), jnp.float32) for _ in range(_NSLOT)]
          + [pltpu.SemaphoreType.DMA for _ in range(_NSLOT)],
    )
    def k(rowt_hbm, colt_hbm, ri_hbm, ci_hbm, out_hbm,
          rowt_v, colt_v, ri_v, ci_v, *slots):
        wid = lax.axis_index("s") * _NC + lax.axis_index("c")
        base = wid * W
        rows = slots[:_NSLOT]
        wsem = slots[_NSLOT:]

        pltpu.sync_copy(rowt_hbm, rowt_v)
        pltpu.sync_copy(colt_hbm, colt_v)
        pltpu.sync_copy(ri_hbm.at[pl.ds(base, W)], ri_v)
        pltpu.sync_copy(ci_hbm.at[pl.ds(base, W)], ci_v)

        def compute_chunk(i, b):
            buf = rows[b]

            def per_group(t, carry):
                g = i * _CHUNK + t * _L
                rv = jnp.clip(ri_v[pl.ds(g, _L)], 0, n_rows - 1) * d_row
                cv = jnp.clip(ci_v[pl.ds(g, _L)], 0, n_cols - 1) * d_col
                for j in range(_L):
                    ro = rv[j]
                    co = cv[j]
                    row = t * _L + j
                    for s in range(d_row // _L):
                        buf[row, pl.ds(s * _L, _L)] = rowt_v[pl.ds(ro + s * _L, _L)]
                    for s in range(d_col // _L):
                        buf[row, pl.ds(d_row + s * _L, _L)] = (
                            colt_v[pl.ds(co + s * _L, _L)])
                return carry

            lax.fori_loop(0, _CHUNK // _L, per_group, 0)

        def w_fire(i, b):
            pltpu.async_copy(rows[b], out_hbm.at[pl.ds(base + i * _CHUNK, _CHUNK)], wsem[b])

        def w_wait(b):
            pltpu.make_async_copy(
                rows[b], out_hbm.at[pl.ds(base, _CHUNK)], wsem[b]
            ).wait()

        for i in range(_NSLOT):
            compute_chunk(i, i)
            w_fire(i, i)

        def steady(kk, carry):
            i0 = _NSLOT + _NSLOT * kk
            for d in range(_NSLOT):
                i = i0 + d
                b = d
                w_wait(b)
                compute_chunk(i, b)
                w_fire(i, b)
            return carry

        lax.fori_loop(0, (n_chunks - _NSLOT) // _NSLOT, steady, 0)

        for b in range(_NSLOT):
            w_wait(b)

    return k


def kernel(row_indices, col_indices, row_pe, col_pe):
    R, Dr = row_pe.shape
    C, Dc = col_pe.shape
    D = Dr + Dc
    shp = row_indices.shape
    B = row_indices.size

    ri = row_indices.reshape(B)
    ci = col_indices.reshape(B)
    rowt = row_pe.reshape(R * Dr)
    colt = col_pe.reshape(C * Dc)

    out = _make_sc_gather(B, D, R, C, Dr)(rowt, colt, ri, ci)
    return out.reshape(shp + (D,))


# speedup vs baseline: 2.2466x; 2.2466x over previous
import functools

import jax
import jax.numpy as jnp
from jax import lax
from jax.experimental import pallas as pl
from jax.experimental.pallas import tpu as pltpu
from jax.experimental.pallas import tpu_sc as plsc

_NC = 2
_NS = 16
_NW = _NC * _NS
_L = 16

_CHUNK = 128
_NSLOT = 4


def _make_sc_gather(B, D, n_rows, n_cols):
    W = B // _NW
    n_chunks = W // _CHUNK
    assert W % _CHUNK == 0
    assert (n_chunks - _NSLOT) % _NSLOT == 0 and n_chunks >= 2 * _NSLOT

    mesh = plsc.VectorSubcoreMesh(core_axis_name="c", subcore_axis_name="s")

    @functools.partial(
        pl.kernel,
        out_type=jax.ShapeDtypeStruct((B, D), jnp.float32),
        mesh=mesh,
        scratch_types=[
            pltpu.VMEM((W,), jnp.int32),
            pltpu.VMEM((W,), jnp.int32),
        ] + [pltpu.VMEM((_CHUNK,), jnp.int32) for _ in range(_NSLOT)]
          + [pltpu.VMEM((_CHUNK, D), jnp.float32) for _ in range(_NSLOT)]
          + [pltpu.SemaphoreType.DMA for _ in range(2 * _NSLOT)],
    )
    def k(table_hbm, ri_hbm, ci_hbm, out_hbm, ri_v, ci_v, *slots):
        wid = lax.axis_index("s") * _NC + lax.axis_index("c")
        base = wid * W
        fi = slots[:_NSLOT]
        rows = slots[_NSLOT:2 * _NSLOT]
        gsem = slots[2 * _NSLOT:3 * _NSLOT]
        wsem = slots[3 * _NSLOT:]

        pltpu.sync_copy(ri_hbm.at[pl.ds(base, W)], ri_v)
        pltpu.sync_copy(ci_hbm.at[pl.ds(base, W)], ci_v)

        def fuse_chunk(i, b):
            for t in range(_CHUNK // _L):
                sl = pl.ds(i * _CHUNK + t * _L, _L)
                r = jnp.clip(ri_v[sl], 0, n_rows - 1)
                cc = jnp.clip(ci_v[sl], 0, n_cols - 1)
                fi[b][pl.ds(t * _L, _L)] = r * n_cols + cc

        def g_fire(i, b):
            fuse_chunk(i, b)
            pltpu.async_copy(table_hbm.at[fi[b]], rows[b], gsem[b])

        def g_wait(b):
            pltpu.make_async_copy(table_hbm.at[fi[b]], rows[b], gsem[b]).wait()

        def w_fire(i, b):
            pltpu.async_copy(rows[b], out_hbm.at[pl.ds(base + i * _CHUNK, _CHUNK)], wsem[b])

        def w_wait(b):
            pltpu.make_async_copy(
                rows[b], out_hbm.at[pl.ds(base, _CHUNK)], wsem[b]
            ).wait()

        for j in range(_NSLOT - 1):
            g_fire(j, j)
        g_wait(0)
        w_fire(0, 0)
        g_fire(_NSLOT - 1, _NSLOT - 1)

        n_steady = n_chunks - _NSLOT

        def steady(kk, carry):
            i0 = 1 + _NSLOT * kk
            for d in range(_NSLOT):
                i = i0 + d
                b = (1 + d) % _NSLOT
                pb = d % _NSLOT
                g_wait(b)
                w_fire(i, b)
                w_wait(pb)
                g_fire(i + _NSLOT - 1, pb)
            return carry

        lax.fori_loop(0, n_steady // _NSLOT, steady, 0)

        for j in range(_NSLOT - 1, 0, -1):
            i = n_chunks - j
            b = i % _NSLOT
            g_wait(b)
            w_fire(i, b)
            w_wait((i - 1) % _NSLOT)
        w_wait((n_chunks - 1) % _NSLOT)

    return k


def kernel(row_indices, col_indices, row_pe, col_pe):
    R, Dr = row_pe.shape
    C, Dc = col_pe.shape
    D = Dr + Dc
    shp = row_indices.shape
    B = row_indices.size

    fused_table = jnp.concatenate(
        [
            jnp.broadcast_to(row_pe[:, None, :], (R, C, Dr)),
            jnp.broadcast_to(col_pe[None, :, :], (R, C, Dc)),
        ],
        axis=-1,
    ).reshape(R * C, D)

    ri = row_indices.reshape(B)
    ci = col_indices.reshape(B)

    out = _make_sc_gather(B, D, R, C)(fused_table, ri, ci)
    return out.reshape(shp + (D,))
